# baseline (speedup 1.0000x reference)
import functools

import jax
import jax.numpy as jnp
from jax import lax
from jax.experimental import pallas as pl
from jax.experimental.pallas import tpu as pltpu
from jax.experimental.pallas import tpu_sc as plsc

_N = 8192 * 4096
_NC, _NS, _L = 2, 16, 16
_NW = _NC * _NS
_PER_W = _N // _NW
_CHUNK = 16384
_NCHUNK = _PER_W // _CHUNK

_mesh = plsc.VectorSubcoreMesh(core_axis_name="c", subcore_axis_name="s")


@functools.partial(
    pl.kernel,
    mesh=_mesh,
    out_type=jax.ShapeDtypeStruct((_N,), jnp.float32),
    scratch_types=[pltpu.VMEM((_CHUNK,), jnp.float32)],
)
def _sc_mask_set(x_hbm, out_hbm, buf):
    wid = lax.axis_index("s") * _NC + lax.axis_index("c")
    base = wid * _PER_W

    def chunk_body(ci, carry):
        off = base + ci * _CHUNK
        pltpu.sync_copy(x_hbm.at[pl.ds(off, _CHUNK)], buf)

        def vec_body(i, c):
            v = buf[pl.ds(i * _L, _L)]
            buf[pl.ds(i * _L, _L)] = jnp.where(v != 0.0, jnp.float32(2.0), v)
            return c

        lax.fori_loop(0, _CHUNK // _L, vec_body, 0)
        pltpu.sync_copy(buf, out_hbm.at[pl.ds(off, _CHUNK)])
        return carry

    lax.fori_loop(0, _NCHUNK, chunk_body, 0)


def kernel(x):
    return _sc_mask_set(x.reshape(-1)).reshape(x.shape)


# speedup vs baseline: 1.8700x; 1.8700x over previous
import functools

import jax
import jax.numpy as jnp
from jax import lax
from jax.experimental import pallas as pl
from jax.experimental.pallas import tpu as pltpu
from jax.experimental.pallas import tpu_sc as plsc

_N = 8192 * 4096
_NC, _NS, _L = 2, 16, 16
_NW = _NC * _NS
_PER_W = _N // _NW
_CHUNK = 16384
_NCHUNK = _PER_W // _CHUNK
_NPAIR = _NCHUNK // 2

_mesh = plsc.VectorSubcoreMesh(core_axis_name="c", subcore_axis_name="s")


@functools.partial(
    pl.kernel,
    mesh=_mesh,
    out_type=jax.ShapeDtypeStruct((_N,), jnp.float32),
    scratch_types=[
        pltpu.VMEM((2, _CHUNK), jnp.float32),
        pltpu.VMEM((2, _CHUNK), jnp.float32),
        pltpu.SemaphoreType.DMA,
        pltpu.SemaphoreType.DMA,
        pltpu.SemaphoreType.DMA,
        pltpu.SemaphoreType.DMA,
    ],
)
def _sc_mask_set(x_hbm, out_hbm, ibuf, obuf, isem0, isem1, osem0, osem1):
    wid = lax.axis_index("s") * _NC + lax.axis_index("c")
    base = wid * _PER_W
    isems = (isem0, isem1)
    osems = (osem0, osem1)

    for b in range(2):
        pltpu.make_async_copy(
            x_hbm.at[pl.ds(base + b * _CHUNK, _CHUNK)], ibuf.at[b], isems[b]
        ).start()

    def pair_body(cp, carry):
        for b in range(2):
            ci = cp * 2 + b
            off = base + ci * _CHUNK
            inb = ibuf.at[b]
            outb = obuf.at[b]
            pltpu.make_async_copy(
                x_hbm.at[pl.ds(off, _CHUNK)], inb, isems[b]
            ).wait()
            @pl.when(ci >= 2)
            def _():
                pltpu.make_async_copy(
                    outb, out_hbm.at[pl.ds(off - 2 * _CHUNK, _CHUNK)], osems[b]
                ).wait()

            @plsc.parallel_loop(0, _CHUNK, 16, unroll=8)
            def _(i):
                v = inb[pl.ds(i, 16)]
                outb[pl.ds(i, 16)] = jnp.where(v != 0.0, jnp.float32(2.0), v)

            pltpu.make_async_copy(
                outb, out_hbm.at[pl.ds(off, _CHUNK)], osems[b]
            ).start()

            @pl.when(ci + 2 < _NCHUNK)
            def _():
                pltpu.make_async_copy(
                    x_hbm.at[pl.ds(off + 2 * _CHUNK, _CHUNK)], inb, isems[b]
                ).start()

        return carry

    lax.fori_loop(0, _NPAIR, pair_body, 0)

    for b in range(2):
        ci = _NCHUNK - 2 + b
        pltpu.make_async_copy(
            obuf.at[b], out_hbm.at[pl.ds(base + ci * _CHUNK, _CHUNK)], osems[b]
        ).wait()


def kernel(x):
    return _sc_mask_set(x.reshape(-1)).reshape(x.shape)


# speedup vs baseline: 6.1892x; 3.3097x over previous
import functools

import jax
import jax.numpy as jnp
from jax import lax
from jax.experimental import pallas as pl
from jax.experimental.pallas import tpu as pltpu
from jax.experimental.pallas import tpu_sc as plsc

_M, _D = 8192, 4096
_NC, _NS, _L = 2, 16, 16
_NW = _NC * _NS
_ROWS_W = _M // _NW
_SLAB = 8
_NSLAB = _ROWS_W // _SLAB

_mesh = plsc.VectorSubcoreMesh(core_axis_name="c", subcore_axis_name="s")


@functools.partial(
    pl.kernel,
    mesh=_mesh,
    out_type=jax.ShapeDtypeStruct((_M, _D), jnp.float32),
    scratch_types=[
        pltpu.VMEM((3, _SLAB, _D), jnp.float32),
        pltpu.SemaphoreType.DMA,
        pltpu.SemaphoreType.DMA,
        pltpu.SemaphoreType.DMA,
        pltpu.SemaphoreType.DMA,
        pltpu.SemaphoreType.DMA,
        pltpu.SemaphoreType.DMA,
    ],
    compiler_params=pltpu.CompilerParams(use_tc_tiling_on_sc=True),
)
def _sc_mask_set(x_hbm, out_hbm, buf, i0, i1, i2, o0, o1, o2):
    wid = lax.axis_index("s") * _NC + lax.axis_index("c")
    base = wid * _ROWS_W
    isems = (i0, i1, i2)
    osems = (o0, o1, o2)

    def in_copy(ci, b):
        return pltpu.make_async_copy(
            x_hbm.at[pl.ds(base + ci * _SLAB, _SLAB), :], buf.at[b], isems[b]
        )

    def out_copy(ci, b):
        return pltpu.make_async_copy(
            buf.at[b], out_hbm.at[pl.ds(base + ci * _SLAB, _SLAB), :], osems[b]
        )

    for b in range(3):
        in_copy(b, b).start()

    def tri_body(q, carry):
        for u in range(3):
            ci = q * 3 + u
            in_copy(ci, u).wait()

            for r in range(_SLAB):
                row = buf.at[u].at[r]

                @plsc.parallel_loop(0, _D, 16, unroll=8)
                def _(i):
                    v = row[pl.ds(i, 16)]
                    row[pl.ds(i, 16)] = jnp.where(v != 0.0, jnp.float32(2.0), v)

            out_copy(ci, u).start()

            bp = (u + 2) % 3

            @pl.when(ci >= 1)
            def _():
                out_copy(ci - 1, bp).wait()
                in_copy(ci + 2, bp).start()

        return carry

    lax.fori_loop(0, _NSLAB // 3, tri_body, 0)

    for ci in (_NSLAB - 2, _NSLAB - 1):
        b = ci % 3
        in_copy(ci, b).wait()
        for r in range(_SLAB):
            row = buf.at[b].at[r]

            @plsc.parallel_loop(0, _D, 16, unroll=8)
            def _(i):
                v = row[pl.ds(i, 16)]
                row[pl.ds(i, 16)] = jnp.where(v != 0.0, jnp.float32(2.0), v)

        out_copy(ci, b).start()

    for ci in (_NSLAB - 3, _NSLAB - 2, _NSLAB - 1):
        out_copy(ci, ci % 3).wait()


def kernel(x):
    return _sc_mask_set(x)
